# baseline (speedup 1.0000x reference)
import functools

import jax
import jax.numpy as jnp
import numpy as np
from jax import lax
from jax.experimental import pallas as pl
from jax.experimental.pallas import tpu as pltpu
from jax.experimental.pallas import tpu_sc as plsc

VOCAB = 1000000
DIM = 64
MAX_LEN = 512
BATCH = 4096
SEQ = 200
N_ROWS = BATCH * SEQ


def _make_pe_np(max_len, dim):
    pos = np.arange(max_len, dtype=np.float32)[:, None]
    div = np.exp(np.arange(0, dim, 2, dtype=np.float32) * (-np.log(10000.0) / dim))
    pe = np.zeros((max_len, dim), dtype=np.float32)
    pe[:, 0::2] = np.sin(pos * div)
    pe[:, 1::2] = np.cos(pos * div)
    return pe


_PE_NP = _make_pe_np(MAX_LEN, DIM)[:SEQ][None, :, :]


_INFO = plsc.get_sparse_core_info()
_NC = _INFO.num_cores
_NS = _INFO.num_subcores
_NW = _NC * _NS
_ROWS_PER_W = N_ROWS // _NW
_IDXW = 128
_CHUNK = 1024
_SUB = _CHUNK // _IDXW
_N_ITER = _ROWS_PER_W // _CHUNK


def _gather_body(idx_hbm, table_hbm, out_hbm, idx_v, rows_v, sem):
    wid = lax.axis_index("s") * _NC + lax.axis_index("c")

    def body(g, _):
        base = pl.multiple_of(wid * _ROWS_PER_W + g * _CHUNK, _CHUNK)
        pltpu.sync_copy(idx_hbm.at[pl.ds(pl.multiple_of(base // _IDXW, 8), _SUB)], idx_v)
        copies = [
            pltpu.async_copy(
                table_hbm.at[idx_v.at[j]],
                rows_v.at[pl.ds(j * _IDXW, _IDXW)],
                sem,
            )
            for j in range(_SUB)
        ]
        for c in copies:
            c.wait()
        pltpu.sync_copy(rows_v, out_hbm.at[pl.ds(base, _CHUNK)])
        return 0

    lax.fori_loop(0, _N_ITER, body, 0)


_gather = functools.partial(
    pl.kernel,
    out_type=jax.ShapeDtypeStruct((N_ROWS, DIM), jnp.float32),
    mesh=plsc.VectorSubcoreMesh(core_axis_name="c", subcore_axis_name="s"),
    compiler_params=pltpu.CompilerParams(use_tc_tiling_on_sc=False),
    scratch_types=[
        pltpu.VMEM((_SUB, _IDXW), jnp.int32),
        pltpu.VMEM((_CHUNK, DIM), jnp.float32),
        pltpu.SemaphoreType.DMA,
    ],
)(_gather_body)


_BB = 16


def _tc_body(x_ref, w_ref, pe_ref, o_ref):
    x = x_ref[...].reshape(_BB * SEQ, DIM)
    y = lax.dot_general(
        x, w_ref[...], (((1,), (1,)), ((), ())), preferred_element_type=jnp.float32
    )
    o_ref[...] = y.reshape(_BB, SEQ, DIM) * 16.0 + pe_ref[...]


def _tc_call(x3, W, pe):
    return pl.pallas_call(
        _tc_body,
        out_shape=jax.ShapeDtypeStruct((BATCH, SEQ, DIM), jnp.float32),
        grid=(BATCH // _BB,),
        in_specs=[
            pl.BlockSpec((_BB, SEQ, DIM), lambda i: (i, 0, 0)),
            pl.BlockSpec((DIM, DIM), lambda i: (0, 0)),
            pl.BlockSpec((1, SEQ, DIM), lambda i: (0, 0, 0)),
        ],
        out_specs=pl.BlockSpec((_BB, SEQ, DIM), lambda i: (i, 0, 0)),
    )(x3, W, pe)


def kernel(src, table, W):
    idx2d = src.astype(jnp.int32).reshape(N_ROWS // _IDXW, _IDXW)
    gathered = _gather(idx2d, table)
    x3 = gathered.reshape(BATCH, SEQ, DIM)
    return _tc_call(x3, W, jnp.asarray(_PE_NP))


# speedup vs baseline: 1.4652x; 1.4652x over previous
import functools

import jax
import jax.numpy as jnp
import numpy as np
from jax import lax
from jax.experimental import pallas as pl
from jax.experimental.pallas import tpu as pltpu
from jax.experimental.pallas import tpu_sc as plsc

VOCAB = 1000000
DIM = 64
BATCH = 4096
SEQ = 200
N_ROWS = BATCH * SEQ
HALF = BATCH // 2


_INFO = plsc.get_sparse_core_info()
_NC = _INFO.num_cores
_NS = _INFO.num_subcores
_NW = _NC * _NS
_ROWS_PER_W = N_ROWS // _NW
_IDXW = 128
_CHUNK = 1024
_SUB = _CHUNK // _IDXW
_N_ITER = _ROWS_PER_W // _CHUNK


def _gather_body(idx_hbm, table_hbm, out_hbm, idx_v, rows_v, sem):
    wid = lax.axis_index("s") * _NC + lax.axis_index("c")

    def body(g, _):
        base = pl.multiple_of(wid * _ROWS_PER_W + g * _CHUNK, _CHUNK)
        pltpu.sync_copy(idx_hbm.at[pl.ds(pl.multiple_of(base // _IDXW, 8), _SUB)], idx_v)
        copies = [
            pltpu.async_copy(
                table_hbm.at[idx_v.at[j]],
                rows_v.at[pl.ds(j * _IDXW, _IDXW)],
                sem,
            )
            for j in range(_SUB)
        ]
        for c in copies:
            c.wait()
        pltpu.sync_copy(rows_v, out_hbm.at[pl.ds(base, _CHUNK)])
        return 0

    lax.fori_loop(0, _N_ITER, body, 0)


_gather = functools.partial(
    pl.kernel,
    out_type=jax.ShapeDtypeStruct((N_ROWS, DIM), jnp.float32),
    mesh=plsc.VectorSubcoreMesh(core_axis_name="c", subcore_axis_name="s"),
    compiler_params=pltpu.CompilerParams(use_tc_tiling_on_sc=False),
    scratch_types=[
        pltpu.VMEM((_SUB, _IDXW), jnp.int32),
        pltpu.VMEM((_CHUNK, DIM), jnp.float32),
        pltpu.SemaphoreType.DMA,
    ],
)(_gather_body)


def _tc_body(g_ref, wd_ref, o_ref):
    l = pl.program_id(0)
    p = g_ref[0]
    z = lax.dot_general(
        wd_ref[...], p, (((1,), (1,)), ((), ())), preferred_element_type=jnp.float32
    )
    di = lax.broadcasted_iota(jnp.int32, (DIM, HALF), 0)
    half_idx = (di // 2).astype(jnp.float32)
    ang = jnp.exp(half_idx * jnp.float32(-2.0 * np.log(10000.0) / DIM)) * jnp.float32(
        l
    ).astype(jnp.float32)
    pe = jnp.where(di % 2 == 0, jnp.sin(ang), jnp.cos(ang))
    o_ref[0, :, 0:HALF] = z[0:DIM] + pe
    o_ref[0, :, HALF:BATCH] = z[DIM : 2 * DIM] + pe


def _tc_call(g3, Wd):
    return pl.pallas_call(
        _tc_body,
        out_shape=jax.ShapeDtypeStruct((SEQ, DIM, BATCH), jnp.float32),
        grid=(SEQ,),
        in_specs=[
            pl.BlockSpec((1, HALF, 2 * DIM), lambda i: (i, 0, 0)),
            pl.BlockSpec((2 * DIM, 2 * DIM), lambda i: (0, 0)),
        ],
        out_specs=pl.BlockSpec((1, DIM, BATCH), lambda i: (i, 0, 0)),
    )(g3, Wd)


def kernel(src, table, W):
    srcT = src.astype(jnp.int32).T
    srci = jnp.stack([srcT[:, :HALF], srcT[:, HALF:]], axis=-1).reshape(SEQ, BATCH)
    idx2d = srci.reshape(N_ROWS // _IDXW, _IDXW)
    gathered = _gather(idx2d, table)
    g3 = gathered.reshape(SEQ, HALF, 2 * DIM)
    Wd = (
        jnp.zeros((2 * DIM, 2 * DIM), jnp.float32)
        .at[:DIM, :DIM]
        .set(W)
        .at[DIM:, DIM:]
        .set(W)
        * 16.0
    )
    out3 = _tc_call(g3, Wd)
    return jnp.transpose(out3, (2, 0, 1))


# speedup vs baseline: 1.9126x; 1.3053x over previous
import functools

import jax
import jax.numpy as jnp
import numpy as np
from jax import lax
from jax.experimental import pallas as pl
from jax.experimental.pallas import tpu as pltpu
from jax.experimental.pallas import tpu_sc as plsc

VOCAB = 1000000
DIM = 64
BATCH = 4096
SEQ = 200
N_ROWS = BATCH * SEQ
HALF = BATCH // 2


_INFO = plsc.get_sparse_core_info()
_NC = _INFO.num_cores
_NS = _INFO.num_subcores
_NW = _NC * _NS
_ROWS_PER_W = N_ROWS // _NW
_IDXW = 128
_CHUNK = 1024
_SUB = _CHUNK // _IDXW
_N_ITER = _ROWS_PER_W // _CHUNK
_HC = _CHUNK // 2
_HB = _HC // _IDXW


def _gather_body(src4_hbm, table_hbm, out_hbm, a_st, b_st, idx_v, rows_v, sem):
    wid = lax.axis_index("s") * _NC + lax.axis_index("c")
    lanes = lax.iota(jnp.int32, 16)

    def body(g, _):
        base = pl.multiple_of(wid * _ROWS_PER_W + g * _CHUNK, _CHUNK)
        l = base // BATCH
        m0 = base % BATCH
        t = l // 8
        s = l % 8
        ca = m0 // 256
        fetches = [
            pltpu.async_copy(
                src4_hbm.at[t, ca + i, s], a_st.at[pl.ds(i * _IDXW, _IDXW)], sem
            )
            for i in range(_HB)
        ] + [
            pltpu.async_copy(
                src4_hbm.at[t, 16 + ca + i, s], b_st.at[pl.ds(i * _IDXW, _IDXW)], sem
            )
            for i in range(_HB)
        ]
        for c in fetches:
            c.wait()
        for j in range(_SUB):
            for u in range(4):
                off = 128 * j + 32 * u + 2 * lanes
                qa = 64 * j + 16 * u
                plsc.store_scatter(idx_v, [off], a_st[pl.ds(qa, 16)] * 2)
                plsc.store_scatter(idx_v, [off + 1], b_st[pl.ds(qa, 16)] * 2)
        gathers = [
            pltpu.async_copy(
                table_hbm.at[idx_v.at[pl.ds(j * _IDXW, _IDXW)]],
                rows_v.at[pl.ds(j * _IDXW, _IDXW)],
                sem,
            )
            for j in range(_SUB)
        ]
        for c in gathers:
            c.wait()
        pltpu.sync_copy(rows_v, out_hbm.at[pl.ds(base, _CHUNK)])
        return 0

    lax.fori_loop(0, _N_ITER, body, 0)


_gather = functools.partial(
    pl.kernel,
    out_type=jax.ShapeDtypeStruct((N_ROWS, DIM), jnp.float32),

    mesh=plsc.VectorSubcoreMesh(core_axis_name="c", subcore_axis_name="s"),
    compiler_params=pltpu.CompilerParams(
        use_tc_tiling_on_sc=False, needs_layout_passes=False
    ),
    scratch_types=[
        pltpu.VMEM((_HC,), jnp.int32),
        pltpu.VMEM((_HC,), jnp.int32),
        pltpu.VMEM((_CHUNK,), jnp.int32),
        pltpu.VMEM((_CHUNK, DIM), jnp.float32),
        pltpu.SemaphoreType.DMA,
    ],
)(_gather_body)


_NB = 2048
_EYE = np.eye(DIM, dtype=np.float32)


def _stage_body(tT_ref, eye_ref, o_ref):
    t = tT_ref[...]
    at = lax.dot_general(
        t, eye_ref[...], (((0,), (0,)), ((), ())), preferred_element_type=jnp.float32
    )
    o_ref[:, 0:DIM] = at
    o_ref[:, DIM : 2 * DIM] = jnp.zeros((_NB, DIM), jnp.float32)


def _stage_call(tT, eye):
    grid = (VOCAB + _NB - 1) // _NB
    return pl.pallas_call(
        _stage_body,
        out_shape=jax.ShapeDtypeStruct((VOCAB, 2 * DIM), jnp.float32),
        grid=(grid,),
        in_specs=[
            pl.BlockSpec((DIM, _NB), lambda i: (0, i)),
            pl.BlockSpec((DIM, DIM), lambda i: (0, 0)),
        ],
        out_specs=pl.BlockSpec((_NB, 2 * DIM), lambda i: (i, 0)),
    )(tT, eye)


def _tc_body(g_ref, wd_ref, o_ref):
    l = pl.program_id(0)
    p = g_ref[0]
    z = lax.dot_general(
        wd_ref[...], p, (((1,), (1,)), ((), ())), preferred_element_type=jnp.float32
    )
    di = lax.broadcasted_iota(jnp.int32, (DIM, HALF), 0)
    half_idx = (di // 2).astype(jnp.float32)
    ang = jnp.exp(half_idx * jnp.float32(-2.0 * np.log(10000.0) / DIM)) * jnp.float32(
        l
    ).astype(jnp.float32)
    pe = jnp.where(di % 2 == 0, jnp.sin(ang), jnp.cos(ang))
    o_ref[0, :, 0:HALF] = z[0:DIM] + pe
    o_ref[0, :, HALF:BATCH] = z[DIM : 2 * DIM] + pe


def _tc_call(g3, Wd):
    return pl.pallas_call(
        _tc_body,
        out_shape=jax.ShapeDtypeStruct((SEQ, DIM, BATCH), jnp.float32),
        grid=(SEQ,),
        in_specs=[
            pl.BlockSpec((1, HALF, 2 * DIM), lambda i: (i, 0, 0)),
            pl.BlockSpec((2 * DIM, 2 * DIM), lambda i: (0, 0)),
        ],
        out_specs=pl.BlockSpec((1, DIM, BATCH), lambda i: (i, 0, 0)),
    )(g3, Wd)


def kernel(src, table, W):
    src4 = (
        src.astype(jnp.int32)
        .T.reshape(SEQ // 8, 8, BATCH // _IDXW, _IDXW)
        .transpose(0, 2, 1, 3)
    )
    t_pad = _stage_call(table.T, jnp.asarray(_EYE))
    table_lin = t_pad.reshape(2 * VOCAB, DIM)
    gathered = _gather(src4, table_lin)
    g3 = gathered.reshape(SEQ, HALF, 2 * DIM)
    Wd = (
        jnp.zeros((2 * DIM, 2 * DIM), jnp.float32)
        .at[:DIM, :DIM]
        .set(W)
        .at[DIM:, DIM:]
        .set(W)
        * 16.0
    )
    out3 = _tc_call(g3, Wd)
    return jnp.transpose(out3, (2, 0, 1))


# speedup vs baseline: 2.3375x; 1.2222x over previous
import functools

import jax
import jax.numpy as jnp
import numpy as np
from jax import lax
from jax.experimental import pallas as pl
from jax.experimental.pallas import tpu as pltpu
from jax.experimental.pallas import tpu_sc as plsc

VOCAB = 1000000
DIM = 64
BATCH = 4096
SEQ = 200
N_ROWS = BATCH * SEQ
HALF = BATCH // 2


_INFO = plsc.get_sparse_core_info()
_NC = _INFO.num_cores
_NS = _INFO.num_subcores
_NW = _NC * _NS
_ROWS_PER_W = N_ROWS // _NW
_IDXW = 128
_CHUNK = 1024
_SUB = _CHUNK // _IDXW
_N_ITER = _ROWS_PER_W // _CHUNK
_HC = _CHUNK // 2
_HB = _HC // _IDXW


def _gather_body(src4_hbm, table_hbm, out_hbm, a_st, b_st, idx_v, rows_v, sem):
    wid = lax.axis_index("s") * _NC + lax.axis_index("c")
    lanes = lax.iota(jnp.int32, 16)

    def body(g, _):
        base = pl.multiple_of(wid * _ROWS_PER_W + g * _CHUNK, _CHUNK)
        l = base // BATCH
        m0 = base % BATCH
        t = l // 8
        s = l % 8
        ca = m0 // 256
        fetches = [
            pltpu.async_copy(
                src4_hbm.at[t, ca + i, s], a_st.at[pl.ds(i * _IDXW, _IDXW)], sem
            )
            for i in range(_HB)
        ] + [
            pltpu.async_copy(
                src4_hbm.at[t, 16 + ca + i, s], b_st.at[pl.ds(i * _IDXW, _IDXW)], sem
            )
            for i in range(_HB)
        ]
        for c in fetches:
            c.wait()
        for j in range(_SUB):
            for u in range(4):
                off = 128 * j + 32 * u + 2 * lanes
                qa = 64 * j + 16 * u
                av = a_st[pl.ds(qa, 16)]
                bv = b_st[pl.ds(qa, 16)]
                ar = jnp.where(av < _K, 2 * av, 2 * av - (2 * _K - 1))
                br = jnp.where(bv < _K, 2 * bv, 2 * bv - (2 * _K - 1))
                plsc.store_scatter(idx_v, [off], ar)
                plsc.store_scatter(idx_v, [off + 1], br)
        gathers = [
            pltpu.async_copy(
                table_hbm.at[idx_v.at[pl.ds(j * _IDXW, _IDXW)]],
                rows_v.at[pl.ds(j * _IDXW, _IDXW)],
                sem,
            )
            for j in range(_SUB)
        ]
        for c in gathers:
            c.wait()
        pltpu.sync_copy(rows_v, out_hbm.at[pl.ds(base, _CHUNK)])
        return 0

    lax.fori_loop(0, _N_ITER, body, 0)


_gather = functools.partial(
    pl.kernel,
    out_type=jax.ShapeDtypeStruct((N_ROWS, DIM), jnp.float32),

    mesh=plsc.VectorSubcoreMesh(core_axis_name="c", subcore_axis_name="s"),
    compiler_params=pltpu.CompilerParams(
        use_tc_tiling_on_sc=False, needs_layout_passes=False
    ),
    scratch_types=[
        pltpu.VMEM((_HC,), jnp.int32),
        pltpu.VMEM((_HC,), jnp.int32),
        pltpu.VMEM((_CHUNK,), jnp.int32),
        pltpu.VMEM((_CHUNK, DIM), jnp.float32),
        pltpu.SemaphoreType.DMA,
    ],
)(_gather_body)


_NB = 2048
_NSTG = 245
_K = _NB * (_NSTG - 1)
_NPACK = _NB * _NSTG
_EYE = np.eye(DIM, dtype=np.float32)


def _stage_body(ta_ref, tb_ref, eye_ref, o_ref):
    eye = eye_ref[...]
    oa = lax.dot_general(
        ta_ref[...], eye, (((0,), (0,)), ((), ())), preferred_element_type=jnp.float32
    )
    ob = lax.dot_general(
        tb_ref[...], eye, (((0,), (0,)), ((), ())), preferred_element_type=jnp.float32
    )
    o_ref[:, 0:DIM] = oa
    o_ref[:, DIM : 2 * DIM] = ob


def _stage_call(tT, eye):
    return pl.pallas_call(
        _stage_body,
        out_shape=jax.ShapeDtypeStruct((_NPACK, 2 * DIM), jnp.float32),
        grid=(_NSTG,),
        in_specs=[
            pl.BlockSpec((DIM, _NB), lambda i: (0, i)),
            pl.BlockSpec((DIM, _NB), lambda i: (0, i + _NSTG - 1)),
            pl.BlockSpec((DIM, DIM), lambda i: (0, 0)),
        ],
        out_specs=pl.BlockSpec((_NB, 2 * DIM), lambda i: (i, 0)),
    )(tT, tT, eye)


def _tc_body(g_ref, wd_ref, o_ref):
    l = pl.program_id(0)
    p = g_ref[0]
    z = lax.dot_general(
        wd_ref[...], p, (((1,), (1,)), ((), ())), preferred_element_type=jnp.float32
    )
    di = lax.broadcasted_iota(jnp.int32, (DIM, HALF), 0)
    half_idx = (di // 2).astype(jnp.float32)
    ang = jnp.exp(half_idx * jnp.float32(-2.0 * np.log(10000.0) / DIM)) * jnp.float32(
        l
    ).astype(jnp.float32)
    pe = jnp.where(di % 2 == 0, jnp.sin(ang), jnp.cos(ang))
    o_ref[0, :, 0:HALF] = z[0:DIM] + pe
    o_ref[0, :, HALF:BATCH] = z[DIM : 2 * DIM] + pe


def _tc_call(g3, Wd):
    return pl.pallas_call(
        _tc_body,
        out_shape=jax.ShapeDtypeStruct((SEQ, DIM, BATCH), jnp.float32),
        grid=(SEQ,),
        in_specs=[
            pl.BlockSpec((1, HALF, 2 * DIM), lambda i: (i, 0, 0)),
            pl.BlockSpec((2 * DIM, 2 * DIM), lambda i: (0, 0)),
        ],
        out_specs=pl.BlockSpec((1, DIM, BATCH), lambda i: (i, 0, 0)),
    )(g3, Wd)


def kernel(src, table, W):
    src4 = (
        src.astype(jnp.int32)
        .T.reshape(SEQ // 8, 8, BATCH // _IDXW, _IDXW)
        .transpose(0, 2, 1, 3)
    )
    t_pack = _stage_call(table.T, jnp.asarray(_EYE))
    table_lin = t_pack.reshape(2 * _NPACK, DIM)
    gathered = _gather(src4, table_lin)
    g3 = gathered.reshape(SEQ, HALF, 2 * DIM)
    Wd = (
        jnp.zeros((2 * DIM, 2 * DIM), jnp.float32)
        .at[:DIM, :DIM]
        .set(W)
        .at[DIM:, DIM:]
        .set(W)
        * 16.0
    )
    out3 = _tc_call(g3, Wd)
    return jnp.transpose(out3, (2, 0, 1))


# speedup vs baseline: 2.8518x; 1.2200x over previous
import functools

import jax
import jax.numpy as jnp
import numpy as np
from jax import lax
from jax.experimental import pallas as pl
from jax.experimental.pallas import tpu as pltpu
from jax.experimental.pallas import tpu_sc as plsc

VOCAB = 1000000
DIM = 64
BATCH = 4096
SEQ = 200
N_ROWS = BATCH * SEQ
HALF = BATCH // 2


_INFO = plsc.get_sparse_core_info()
_NC = _INFO.num_cores
_NS = _INFO.num_subcores
_NW = _NC * _NS
_ROWS_PER_W = N_ROWS // _NW
_IDXW = 128
_CHUNK = 1024
_SUB = _CHUNK // _IDXW
_N_ITER = _ROWS_PER_W // _CHUNK
_HC = _CHUNK // 2
_HB = _HC // _IDXW


def _gather_body(src4_hbm, table_hbm, out_hbm, a_st, b_st, idx_v, rows_v, sem):
    wid = lax.axis_index("s") * _NC + lax.axis_index("c")
    lanes = lax.iota(jnp.int32, 16)

    def body(g, _):
        base = pl.multiple_of(wid * _ROWS_PER_W + g * _CHUNK, _CHUNK)
        l = base // BATCH
        m0 = base % BATCH
        t = l // 8
        s = l % 8
        ca = m0 // 256
        fetches = [
            pltpu.async_copy(
                src4_hbm.at[t, ca + i, s], a_st.at[pl.ds(i * _IDXW, _IDXW)], sem
            )
            for i in range(_HB)
        ] + [
            pltpu.async_copy(
                src4_hbm.at[t, 16 + ca + i, s], b_st.at[pl.ds(i * _IDXW, _IDXW)], sem
            )
            for i in range(_HB)
        ]
        for c in fetches:
            c.wait()
        for j in range(_SUB):
            for u in range(4):
                off = 128 * j + 32 * u + 2 * lanes
                qa = 64 * j + 16 * u
                av = a_st[pl.ds(qa, 16)]
                bv = b_st[pl.ds(qa, 16)]
                ar = jnp.where(av < _K, 2 * av, 2 * av - (2 * _K - 1))
                br = jnp.where(bv < _K, 2 * bv, 2 * bv - (2 * _K - 1))
                plsc.store_scatter(idx_v, [off], ar)
                plsc.store_scatter(idx_v, [off + 1], br)
        gathers = [
            pltpu.async_copy(
                table_hbm.at[idx_v.at[pl.ds(j * _IDXW, _IDXW)]],
                rows_v.at[pl.ds(j * _IDXW, _IDXW)],
                sem,
            )
            for j in range(_SUB)
        ]
        for c in gathers:
            c.wait()
        pltpu.sync_copy(rows_v, out_hbm.at[pl.ds(base, _CHUNK)])
        return 0

    lax.fori_loop(0, _N_ITER, body, 0)


_gather = functools.partial(
    pl.kernel,
    out_type=jax.ShapeDtypeStruct((N_ROWS, DIM), jnp.float32),

    mesh=plsc.VectorSubcoreMesh(core_axis_name="c", subcore_axis_name="s"),
    compiler_params=pltpu.CompilerParams(
        use_tc_tiling_on_sc=False, needs_layout_passes=False
    ),
    scratch_types=[
        pltpu.VMEM((_HC,), jnp.int32),
        pltpu.VMEM((_HC,), jnp.int32),
        pltpu.VMEM((_CHUNK,), jnp.int32),
        pltpu.VMEM((_CHUNK, DIM), jnp.float32),
        pltpu.SemaphoreType.DMA,
    ],
)(_gather_body)


_NB = 4096
_NSTG = 123
_K = _NB * (_NSTG - 1)
_NPACK = _NB * _NSTG
_EYE = np.eye(DIM, dtype=np.float32)


def _stage_body(ta_ref, tb_ref, eye_ref, o_ref):
    eye = eye_ref[...]
    oa = lax.dot_general(
        ta_ref[...], eye, (((0,), (0,)), ((), ())), preferred_element_type=jnp.float32
    )
    ob = lax.dot_general(
        tb_ref[...], eye, (((0,), (0,)), ((), ())), preferred_element_type=jnp.float32
    )
    o_ref[:, 0:DIM] = oa
    o_ref[:, DIM : 2 * DIM] = ob


def _stage_call(tT, eye):
    return pl.pallas_call(
        _stage_body,
        out_shape=jax.ShapeDtypeStruct((_NPACK, 2 * DIM), jnp.float32),
        grid=(_NSTG,),
        in_specs=[
            pl.BlockSpec((DIM, _NB), lambda i: (0, i)),
            pl.BlockSpec((DIM, _NB), lambda i: (0, i + _NSTG - 1)),
            pl.BlockSpec((DIM, DIM), lambda i: (0, 0)),
        ],
        out_specs=pl.BlockSpec((_NB, 2 * DIM), lambda i: (i, 0)),
    )(tT, tT, eye)


_LB = 2


def _pe_cols(l):
    di = lax.broadcasted_iota(jnp.int32, (DIM, HALF), 0)
    half_idx = (di // 2).astype(jnp.float32)
    ang = jnp.exp(half_idx * jnp.float32(-2.0 * np.log(10000.0) / DIM)) * jnp.float32(
        l
    ).astype(jnp.float32)
    return jnp.where(di % 2 == 0, jnp.sin(ang), jnp.cos(ang))


def _tc_body(g_ref, wd_ref, o_ref):
    l0 = pl.program_id(0) * _LB
    p = g_ref[...].reshape(_LB * HALF, 2 * DIM)
    z = lax.dot_general(
        wd_ref[...], p, (((1,), (1,)), ((), ())), preferred_element_type=jnp.float32
    )
    for u in range(_LB):
        pe = _pe_cols(l0 + u)
        zs = z[:, u * HALF : (u + 1) * HALF]
        o_ref[u, :, 0:HALF] = zs[0:DIM] + pe
        o_ref[u, :, HALF:BATCH] = zs[DIM : 2 * DIM] + pe


def _tc_call(g3, Wd):
    return pl.pallas_call(
        _tc_body,
        out_shape=jax.ShapeDtypeStruct((SEQ, DIM, BATCH), jnp.float32),
        grid=(SEQ // _LB,),
        in_specs=[
            pl.BlockSpec((_LB, HALF, 2 * DIM), lambda i: (i, 0, 0)),
            pl.BlockSpec((2 * DIM, 2 * DIM), lambda i: (0, 0)),
        ],
        out_specs=pl.BlockSpec((_LB, DIM, BATCH), lambda i: (i, 0, 0)),
    )(g3, Wd)


def kernel(src, table, W):
    src4 = (
        src.astype(jnp.int32)
        .T.reshape(SEQ // 8, 8, BATCH // _IDXW, _IDXW)
        .transpose(0, 2, 1, 3)
    )
    t_pack = _stage_call(table.T, jnp.asarray(_EYE))
    table_lin = t_pack.reshape(2 * _NPACK, DIM)
    gathered = _gather(src4, table_lin)
    g3 = gathered.reshape(SEQ, HALF, 2 * DIM)
    Wd = (
        jnp.zeros((2 * DIM, 2 * DIM), jnp.float32)
        .at[:DIM, :DIM]
        .set(W)
        .at[DIM:, DIM:]
        .set(W)
        * 16.0
    )
    out3 = _tc_call(g3, Wd)
    return jnp.transpose(out3, (2, 0, 1))


# speedup vs baseline: 2.9550x; 1.0362x over previous
import functools

import jax
import jax.numpy as jnp
import numpy as np
from jax import lax
from jax.experimental import pallas as pl
from jax.experimental.pallas import tpu as pltpu
from jax.experimental.pallas import tpu_sc as plsc

VOCAB = 1000000
DIM = 64
BATCH = 4096
SEQ = 200
N_ROWS = BATCH * SEQ
HALF = BATCH // 2


_INFO = plsc.get_sparse_core_info()
_NC = _INFO.num_cores
_NS = _INFO.num_subcores
_NW = _NC * _NS
_NSH = 5
_SHROWS = N_ROWS // _NSH
_ROWS_PER_W = _SHROWS // _NW
_IDXW = 128
_CHUNK = 1024
_SUB = _CHUNK // _IDXW
_N_ITER = _ROWS_PER_W // _CHUNK
_HC = _CHUNK // 2
_HB = _HC // _IDXW


def _gather_body(shard, src4_hbm, table_hbm, out_hbm, a_st, b_st, idx_v, rows_v, sem):
    wid = lax.axis_index("s") * _NC + lax.axis_index("c")
    lanes = lax.iota(jnp.int32, 16)

    def body(g, _):
        lbase = pl.multiple_of(wid * _ROWS_PER_W + g * _CHUNK, _CHUNK)
        base = pl.multiple_of(shard * _SHROWS + lbase, _CHUNK)
        l = base // BATCH
        m0 = base % BATCH
        t = l // 8
        s = l % 8
        ca = m0 // 256
        fetches = [
            pltpu.async_copy(
                src4_hbm.at[t, ca + i, s], a_st.at[pl.ds(i * _IDXW, _IDXW)], sem
            )
            for i in range(_HB)
        ] + [
            pltpu.async_copy(
                src4_hbm.at[t, 16 + ca + i, s], b_st.at[pl.ds(i * _IDXW, _IDXW)], sem
            )
            for i in range(_HB)
        ]
        for c in fetches:
            c.wait()
        for j in range(_SUB):
            for u in range(4):
                off = 128 * j + 32 * u + 2 * lanes
                qa = 64 * j + 16 * u
                av = a_st[pl.ds(qa, 16)]
                bv = b_st[pl.ds(qa, 16)]
                ar = jnp.where(av < _K, 2 * av, 2 * av - (2 * _K - 1))
                br = jnp.where(bv < _K, 2 * bv, 2 * bv - (2 * _K - 1))
                plsc.store_scatter(idx_v, [off], ar)
                plsc.store_scatter(idx_v, [off + 1], br)
        gathers = [
            pltpu.async_copy(
                table_hbm.at[idx_v.at[pl.ds(j * _IDXW, _IDXW)]],
                rows_v.at[pl.ds(j * _IDXW, _IDXW)],
                sem,
            )
            for j in range(_SUB)
        ]
        for c in gathers:
            c.wait()
        pltpu.sync_copy(rows_v, out_hbm.at[pl.ds(lbase, _CHUNK)])
        return 0

    lax.fori_loop(0, _N_ITER, body, 0)


_GATHERS = [
    functools.partial(
        pl.kernel,
        out_type=jax.ShapeDtypeStruct((_SHROWS, DIM), jnp.float32),
        mesh=plsc.VectorSubcoreMesh(core_axis_name="c", subcore_axis_name="s"),
        compiler_params=pltpu.CompilerParams(
            use_tc_tiling_on_sc=False, needs_layout_passes=False
        ),
        scratch_types=[
            pltpu.VMEM((_HC,), jnp.int32),
            pltpu.VMEM((_HC,), jnp.int32),
            pltpu.VMEM((_CHUNK,), jnp.int32),
            pltpu.VMEM((_CHUNK, DIM), jnp.float32),
            pltpu.SemaphoreType.DMA,
        ],
    )(functools.partial(_gather_body, _s))
    for _s in range(_NSH)
]


_NB = 4096
_NSTG = 123
_K = _NB * (_NSTG - 1)
_NPACK = _NB * _NSTG
_EYE = np.eye(DIM, dtype=np.float32)


def _stage_body(ta_ref, tb_ref, eye_ref, o_ref):
    eye = eye_ref[...]
    oa = lax.dot_general(
        ta_ref[...], eye, (((0,), (0,)), ((), ())), preferred_element_type=jnp.float32
    )
    ob = lax.dot_general(
        tb_ref[...], eye, (((0,), (0,)), ((), ())), preferred_element_type=jnp.float32
    )
    o_ref[:, 0:DIM] = oa
    o_ref[:, DIM : 2 * DIM] = ob


def _stage_call(tT, eye):
    return pl.pallas_call(
        _stage_body,
        out_shape=jax.ShapeDtypeStruct((_NPACK, 2 * DIM), jnp.float32),
        grid=(_NSTG,),
        in_specs=[
            pl.BlockSpec((DIM, _NB), lambda i: (0, i)),
            pl.BlockSpec((DIM, _NB), lambda i: (0, i + _NSTG - 1)),
            pl.BlockSpec((DIM, DIM), lambda i: (0, 0)),
        ],
        out_specs=pl.BlockSpec((_NB, 2 * DIM), lambda i: (i, 0)),
    )(tT, tT, eye)


_LB = 2


def _pe_cols(l):
    di = lax.broadcasted_iota(jnp.int32, (DIM, HALF), 0)
    half_idx = (di // 2).astype(jnp.float32)
    ang = jnp.exp(half_idx * jnp.float32(-2.0 * np.log(10000.0) / DIM)) * jnp.float32(
        l
    ).astype(jnp.float32)
    return jnp.where(di % 2 == 0, jnp.sin(ang), jnp.cos(ang))


_SHL = SEQ // _NSH
_SHG = _SHL // _LB


def _tc_shard_body(shard, g_ref, wd_ref, o_ref):
    l0 = (shard * _SHL) + pl.program_id(0) * _LB
    p = g_ref[...].reshape(_LB * HALF, 2 * DIM)
    z = lax.dot_general(
        wd_ref[...], p, (((1,), (1,)), ((), ())), preferred_element_type=jnp.float32
    )
    for u in range(_LB):
        pe = _pe_cols(l0 + u)
        zs = z[:, u * HALF : (u + 1) * HALF]
        o_ref[u, :, 0:HALF] = zs[0:DIM] + pe
        o_ref[u, :, HALF:BATCH] = zs[DIM : 2 * DIM] + pe


def _tc_shard_body_acc(shard, acc_ref, g_ref, wd_ref, o_ref):
    _tc_shard_body(shard, g_ref, wd_ref, o_ref)


def _tc_call_shard(shard, acc, g3s, Wd):
    out_shape = jax.ShapeDtypeStruct((SEQ, DIM, BATCH), jnp.float32)
    out_spec = pl.BlockSpec(
        (_LB, DIM, BATCH), lambda i, s=shard: (s * _SHG + i, 0, 0)
    )
    g_spec = pl.BlockSpec((_LB, HALF, 2 * DIM), lambda i: (i, 0, 0))
    w_spec = pl.BlockSpec((2 * DIM, 2 * DIM), lambda i: (0, 0))
    if acc is None:
        return pl.pallas_call(
            functools.partial(_tc_shard_body, shard),
            out_shape=out_shape,
            grid=(_SHG,),
            in_specs=[g_spec, w_spec],
            out_specs=out_spec,
        )(g3s, Wd)
    return pl.pallas_call(
        functools.partial(_tc_shard_body_acc, shard),
        out_shape=out_shape,
        grid=(_SHG,),
        in_specs=[pl.BlockSpec(memory_space=pltpu.HBM), g_spec, w_spec],
        out_specs=out_spec,
        input_output_aliases={0: 0},
    )(acc, g3s, Wd)


def kernel(src, table, W):
    src4 = (
        src.astype(jnp.int32)
        .T.reshape(SEQ // 8, 8, BATCH // _IDXW, _IDXW)
        .transpose(0, 2, 1, 3)
    )
    t_pack = _stage_call(table.T, jnp.asarray(_EYE))
    table_lin = t_pack.reshape(2 * _NPACK, DIM)
    Wd = (
        jnp.zeros((2 * DIM, 2 * DIM), jnp.float32)
        .at[:DIM, :DIM]
        .set(W)
        .at[DIM:, DIM:]
        .set(W)
        * 16.0
    )
    acc = None
    for s in range(_NSH):
        gathered = _GATHERS[s](src4, table_lin)
        g3s = gathered.reshape(_SHL, HALF, 2 * DIM)
        acc = _tc_call_shard(s, acc, g3s, Wd)
    return jnp.transpose(acc, (2, 0, 1))


# speedup vs baseline: 3.1584x; 1.0688x over previous
import functools

import jax
import jax.numpy as jnp
import numpy as np
from jax import lax
from jax.experimental import pallas as pl
from jax.experimental.pallas import tpu as pltpu
from jax.experimental.pallas import tpu_sc as plsc

VOCAB = 1000000
DIM = 64
BATCH = 4096
SEQ = 200
N_ROWS = BATCH * SEQ
HALF = BATCH // 2


_INFO = plsc.get_sparse_core_info()
_NC = _INFO.num_cores
_NS = _INFO.num_subcores
_NW = _NC * _NS
_NSH = 5
_SHROWS = N_ROWS // _NSH
_ROWS_PER_W = _SHROWS // _NW
_IDXW = 128
_CHUNK = 1024
_SUB = _CHUNK // _IDXW
_N_ITER = _ROWS_PER_W // _CHUNK
_HC = _CHUNK // 2
_HB = _HC // _IDXW


def _gather_body(shard, src4_hbm, table_hbm, out_hbm, a_st, b_st, idx_v, rows_v, sem):
    wid = lax.axis_index("s") * _NC + lax.axis_index("c")
    lanes = lax.iota(jnp.int32, 16)

    def body(g, _):
        lbase = pl.multiple_of(wid * _ROWS_PER_W + g * _CHUNK, _CHUNK)
        base = pl.multiple_of(shard * _SHROWS + lbase, _CHUNK)
        l = base // BATCH
        m0 = base % BATCH
        t = l // 8
        s = l % 8
        ca = m0 // 256
        fetches = [
            pltpu.async_copy(
                src4_hbm.at[t, ca + i, s], a_st.at[pl.ds(i * _IDXW, _IDXW)], sem
            )
            for i in range(_HB)
        ] + [
            pltpu.async_copy(
                src4_hbm.at[t, 16 + ca + i, s], b_st.at[pl.ds(i * _IDXW, _IDXW)], sem
            )
            for i in range(_HB)
        ]
        for c in fetches:
            c.wait()
        for j in range(_SUB):
            for u in range(4):
                off = 128 * j + 32 * u + 2 * lanes
                qa = 64 * j + 16 * u
                av = a_st[pl.ds(qa, 16)]
                bv = b_st[pl.ds(qa, 16)]
                ar = jnp.where(av < _K, 2 * av, 2 * av - (2 * _K - 1))
                br = jnp.where(bv < _K, 2 * bv, 2 * bv - (2 * _K - 1))
                plsc.store_scatter(idx_v, [off], ar)
                plsc.store_scatter(idx_v, [off + 1], br)
        gathers = [
            pltpu.async_copy(
                table_hbm.at[idx_v.at[pl.ds(j * _IDXW, _IDXW)]],
                rows_v.at[pl.ds(j * _IDXW, _IDXW)],
                sem,
            )
            for j in range(_SUB)
        ]
        for c in gathers:
            c.wait()
        pltpu.sync_copy(rows_v, out_hbm.at[pl.ds(lbase, _CHUNK)])
        return 0

    lax.fori_loop(0, _N_ITER, body, 0)


_GATHERS = [
    functools.partial(
        pl.kernel,
        out_type=jax.ShapeDtypeStruct((_SHROWS, DIM), jnp.float32),
        mesh=plsc.VectorSubcoreMesh(core_axis_name="c", subcore_axis_name="s"),
        compiler_params=pltpu.CompilerParams(
            use_tc_tiling_on_sc=False, needs_layout_passes=False
        ),
        scratch_types=[
            pltpu.VMEM((_HC,), jnp.int32),
            pltpu.VMEM((_HC,), jnp.int32),
            pltpu.VMEM((_CHUNK,), jnp.int32),
            pltpu.VMEM((_CHUNK, DIM), jnp.float32),
            pltpu.SemaphoreType.DMA,
        ],
    )(functools.partial(_gather_body, _s))
    for _s in range(_NSH)
]


_NB = 8192
_NSTG = 62
_K = _NB * (_NSTG - 1)
_NPACK = _NB * _NSTG
_EYE = np.eye(DIM, dtype=np.float32)


def _stage_body(ta_ref, tb_ref, eye_ref, o_ref):
    eye = eye_ref[...]
    oa = lax.dot_general(
        ta_ref[...], eye, (((0,), (0,)), ((), ())), preferred_element_type=jnp.float32
    )
    ob = lax.dot_general(
        tb_ref[...], eye, (((0,), (0,)), ((), ())), preferred_element_type=jnp.float32
    )
    o_ref[:, 0:DIM] = oa
    o_ref[:, DIM : 2 * DIM] = ob


def _stage_call(tT, eye):
    return pl.pallas_call(
        _stage_body,
        out_shape=jax.ShapeDtypeStruct((_NPACK, 2 * DIM), jnp.float32),
        grid=(_NSTG,),
        in_specs=[
            pl.BlockSpec((DIM, _NB), lambda i: (0, i)),
            pl.BlockSpec((DIM, _NB), lambda i: (0, i + _NSTG - 1)),
            pl.BlockSpec((DIM, DIM), lambda i: (0, 0)),
        ],
        out_specs=pl.BlockSpec((_NB, 2 * DIM), lambda i: (i, 0)),
    )(tT, tT, eye)


_LB = 4


def _pe_cols(l):
    di = lax.broadcasted_iota(jnp.int32, (DIM, HALF), 0)
    half_idx = (di // 2).astype(jnp.float32)
    ang = jnp.exp(half_idx * jnp.float32(-2.0 * np.log(10000.0) / DIM)) * jnp.float32(
        l
    ).astype(jnp.float32)
    return jnp.where(di % 2 == 0, jnp.sin(ang), jnp.cos(ang))


_SHL = SEQ // _NSH
_SHG = _SHL // _LB


def _tc_shard_body(shard, g_ref, wd_ref, o_ref):
    l0 = (shard * _SHL) + pl.program_id(0) * _LB
    p = g_ref[...].reshape(_LB * HALF, 2 * DIM)
    z = lax.dot_general(
        wd_ref[...], p, (((1,), (1,)), ((), ())), preferred_element_type=jnp.float32
    )
    for u in range(_LB):
        pe = _pe_cols(l0 + u)
        zs = z[:, u * HALF : (u + 1) * HALF]
        o_ref[u, :, 0:HALF] = zs[0:DIM] + pe
        o_ref[u, :, HALF:BATCH] = zs[DIM : 2 * DIM] + pe


def _tc_shard_body_acc(shard, acc_ref, g_ref, wd_ref, o_ref):
    _tc_shard_body(shard, g_ref, wd_ref, o_ref)


def _tc_call_shard(shard, acc, g3s, Wd):
    out_shape = jax.ShapeDtypeStruct((SEQ, DIM, BATCH), jnp.float32)
    out_spec = pl.BlockSpec(
        (_LB, DIM, BATCH), lambda i, s=shard: (s * _SHG + i, 0, 0)
    )
    g_spec = pl.BlockSpec((_LB, HALF, 2 * DIM), lambda i: (i, 0, 0))
    w_spec = pl.BlockSpec((2 * DIM, 2 * DIM), lambda i: (0, 0))
    if acc is None:
        return pl.pallas_call(
            functools.partial(_tc_shard_body, shard),
            out_shape=out_shape,
            grid=(_SHG,),
            in_specs=[g_spec, w_spec],
            out_specs=out_spec,
        )(g3s, Wd)
    return pl.pallas_call(
        functools.partial(_tc_shard_body_acc, shard),
        out_shape=out_shape,
        grid=(_SHG,),
        in_specs=[pl.BlockSpec(memory_space=pltpu.HBM), g_spec, w_spec],
        out_specs=out_spec,
        input_output_aliases={0: 0},
    )(acc, g3s, Wd)


def kernel(src, table, W):
    src4 = (
        src.astype(jnp.int32)
        .T.reshape(SEQ // 8, 8, BATCH // _IDXW, _IDXW)
        .transpose(0, 2, 1, 3)
    )
    t_pack = _stage_call(table.T, jnp.asarray(_EYE))
    table_lin = t_pack.reshape(2 * _NPACK, DIM)
    Wd = (
        jnp.zeros((2 * DIM, 2 * DIM), jnp.float32)
        .at[:DIM, :DIM]
        .set(W)
        .at[DIM:, DIM:]
        .set(W)
        * 16.0
    )
    acc = None
    for s in range(_NSH):
        gathered = _GATHERS[s](src4, table_lin)
        g3s = gathered.reshape(_SHL, HALF, 2 * DIM)
        acc = _tc_call_shard(s, acc, g3s, Wd)
    return jnp.transpose(acc, (2, 0, 1))


# speedup vs baseline: 3.1742x; 1.0050x over previous
import functools

import jax
import jax.numpy as jnp
import numpy as np
from jax import lax
from jax.experimental import pallas as pl
from jax.experimental.pallas import tpu as pltpu
from jax.experimental.pallas import tpu_sc as plsc

VOCAB = 1000000
DIM = 64
BATCH = 4096
SEQ = 200
N_ROWS = BATCH * SEQ
HALF = BATCH // 2


_INFO = plsc.get_sparse_core_info()
_NC = _INFO.num_cores
_NS = _INFO.num_subcores
_NW = _NC * _NS
_NSH = 5
_SHROWS = N_ROWS // _NSH
_ROWS_PER_W = _SHROWS // _NW
_IDXW = 128
_CHUNK = 1024
_SUB = _CHUNK // _IDXW
_N_ITER = _ROWS_PER_W // _CHUNK
_HC = _CHUNK // 2
_HB = _HC // _IDXW


def _gather_body(shard, src4_hbm, table_hbm, out_hbm, a_st, b_st, idx_v, rows_v, sem):
    wid = lax.axis_index("s") * _NC + lax.axis_index("c")
    lanes = lax.iota(jnp.int32, 16)

    def body(g, _):
        lbase = pl.multiple_of(wid * _ROWS_PER_W + g * _CHUNK, _CHUNK)
        base = pl.multiple_of(shard * _SHROWS + lbase, _CHUNK)
        l = base // BATCH
        m0 = base % BATCH
        t = l // 8
        s = l % 8
        ca = m0 // 256
        fetches = [
            pltpu.async_copy(
                src4_hbm.at[t, ca + i, s], a_st.at[pl.ds(i * _IDXW, _IDXW)], sem
            )
            for i in range(_HB)
        ] + [
            pltpu.async_copy(
                src4_hbm.at[t, 16 + ca + i, s], b_st.at[pl.ds(i * _IDXW, _IDXW)], sem
            )
            for i in range(_HB)
        ]
        for c in fetches:
            c.wait()
        for j in range(_SUB):
            for u in range(4):
                off = 128 * j + 32 * u + 2 * lanes
                qa = 64 * j + 16 * u
                av = a_st[pl.ds(qa, 16)]
                bv = b_st[pl.ds(qa, 16)]
                ar = jnp.where(av < _K, 2 * av, 2 * av - (2 * _K - 1))
                br = jnp.where(bv < _K, 2 * bv, 2 * bv - (2 * _K - 1))
                plsc.store_scatter(idx_v, [off], ar)
                plsc.store_scatter(idx_v, [off + 1], br)
        gathers = [
            pltpu.async_copy(
                table_hbm.at[idx_v.at[pl.ds(j * _IDXW, _IDXW)]],
                rows_v.at[pl.ds(j * _IDXW, _IDXW)],
                sem,
            )
            for j in range(_SUB)
        ]
        for c in gathers:
            c.wait()
        pltpu.sync_copy(rows_v, out_hbm.at[pl.ds(lbase, _CHUNK)])
        return 0

    lax.fori_loop(0, _N_ITER, body, 0)


_GATHERS = [
    functools.partial(
        pl.kernel,
        out_type=jax.ShapeDtypeStruct((_SHROWS, DIM), jnp.float32),
        mesh=plsc.VectorSubcoreMesh(core_axis_name="c", subcore_axis_name="s"),
        compiler_params=pltpu.CompilerParams(
            use_tc_tiling_on_sc=False, needs_layout_passes=False
        ),
        scratch_types=[
            pltpu.VMEM((_HC,), jnp.int32),
            pltpu.VMEM((_HC,), jnp.int32),
            pltpu.VMEM((_CHUNK,), jnp.int32),
            pltpu.VMEM((_CHUNK, DIM), jnp.float32),
            pltpu.SemaphoreType.DMA,
        ],
    )(functools.partial(_gather_body, _s))
    for _s in range(_NSH)
]


_NB = 8192
_NSTG = 62
_K = _NB * (_NSTG - 1)
_NPACK = _NB * _NSTG
_EYE = np.eye(DIM, dtype=np.float32)


def _stage_body(ta_ref, tb_ref, eye_ref, o_ref):
    eye = eye_ref[...]
    oa = lax.dot_general(
        ta_ref[...], eye, (((0,), (0,)), ((), ())), preferred_element_type=jnp.float32
    )
    ob = lax.dot_general(
        tb_ref[...], eye, (((0,), (0,)), ((), ())), preferred_element_type=jnp.float32
    )
    o_ref[:, 0:DIM] = oa
    o_ref[:, DIM : 2 * DIM] = ob


def _stage_call(tT, eye):
    return pl.pallas_call(
        _stage_body,
        out_shape=jax.ShapeDtypeStruct((_NPACK, 2 * DIM), jnp.float32),
        grid=(_NSTG,),
        in_specs=[
            pl.BlockSpec((DIM, _NB), lambda i: (0, i)),
            pl.BlockSpec((DIM, _NB), lambda i: (0, i + _NSTG - 1)),
            pl.BlockSpec((DIM, DIM), lambda i: (0, 0)),
        ],
        out_specs=pl.BlockSpec((_NB, 2 * DIM), lambda i: (i, 0)),
    )(tT, tT, eye)


_LB = 8


def _pe_cols(l):
    di = lax.broadcasted_iota(jnp.int32, (DIM, HALF), 0)
    half_idx = (di // 2).astype(jnp.float32)
    ang = jnp.exp(half_idx * jnp.float32(-2.0 * np.log(10000.0) / DIM)) * jnp.float32(
        l
    ).astype(jnp.float32)
    return jnp.where(di % 2 == 0, jnp.sin(ang), jnp.cos(ang))


_SHL = SEQ // _NSH
_SHG = _SHL // _LB


def _tc_shard_body(shard, g_ref, wd_ref, o_ref):
    l0 = (shard * _SHL) + pl.program_id(0) * _LB
    p = g_ref[...].reshape(_LB * HALF, 2 * DIM)
    z = lax.dot_general(
        wd_ref[...], p, (((1,), (1,)), ((), ())), preferred_element_type=jnp.float32
    )
    for u in range(_LB):
        pe = _pe_cols(l0 + u)
        zs = z[:, u * HALF : (u + 1) * HALF]
        o_ref[u, :, 0:HALF] = zs[0:DIM] + pe
        o_ref[u, :, HALF:BATCH] = zs[DIM : 2 * DIM] + pe


def _tc_shard_body_acc(shard, acc_ref, g_ref, wd_ref, o_ref):
    _tc_shard_body(shard, g_ref, wd_ref, o_ref)


def _tc_call_shard(shard, acc, g3s, Wd):
    out_shape = jax.ShapeDtypeStruct((SEQ, DIM, BATCH), jnp.float32)
    out_spec = pl.BlockSpec(
        (_LB, DIM, BATCH), lambda i, s=shard: (s * _SHG + i, 0, 0)
    )
    g_spec = pl.BlockSpec((_LB, HALF, 2 * DIM), lambda i: (i, 0, 0))
    w_spec = pl.BlockSpec((2 * DIM, 2 * DIM), lambda i: (0, 0))
    if acc is None:
        return pl.pallas_call(
            functools.partial(_tc_shard_body, shard),
            out_shape=out_shape,
            grid=(_SHG,),
            in_specs=[g_spec, w_spec],
            out_specs=out_spec,
        )(g3s, Wd)
    return pl.pallas_call(
        functools.partial(_tc_shard_body_acc, shard),
        out_shape=out_shape,
        grid=(_SHG,),
        in_specs=[pl.BlockSpec(memory_space=pltpu.HBM), g_spec, w_spec],
        out_specs=out_spec,
        input_output_aliases={0: 0},
    )(acc, g3s, Wd)


def kernel(src, table, W):
    src4 = (
        src.astype(jnp.int32)
        .T.reshape(SEQ // 8, 8, BATCH // _IDXW, _IDXW)
        .transpose(0, 2, 1, 3)
    )
    t_pack = _stage_call(table.T, jnp.asarray(_EYE))
    table_lin = t_pack.reshape(2 * _NPACK, DIM)
    Wd = (
        jnp.zeros((2 * DIM, 2 * DIM), jnp.float32)
        .at[:DIM, :DIM]
        .set(W)
        .at[DIM:, DIM:]
        .set(W)
        * 16.0
    )
    acc = None
    for s in range(_NSH):
        gathered = _GATHERS[s](src4, table_lin)
        g3s = gathered.reshape(_SHL, HALF, 2 * DIM)
        acc = _tc_call_shard(s, acc, g3s, Wd)
    return jnp.transpose(acc, (2, 0, 1))
